# baseline (speedup 1.0000x reference)
import functools

import jax
import jax.numpy as jnp
from jax import lax
from jax.experimental import pallas as pl
from jax.experimental.pallas import tpu as pltpu
from jax.experimental.pallas import tpu_sc as plsc

N = 4096
DIM = 1024
E = 8
HID = 1536

BT = 512
NB = N // BT + E
NB_PAD = 128
NPAD = NB * BT

GATE_BG = 512
GATE_NBLK = N // GATE_BG

NC = 2
NS = 16
NW = NC * NS
ROWS_PER_CHUNK = 32


def _gate_body(x_ref, wg_ref, bg_ref, top1_ref, rank_ref, starts_ref,
               bemap_ref, ordmap_ref, seg_ref, base_ref):
    b = pl.program_id(0)

    @pl.when(b == 0)
    def _():
        base_ref[...] = jnp.zeros((1, 16), jnp.float32)

    logits = lax.dot(x_ref[...], wg_ref[...],
                     preferred_element_type=jnp.float32) + bg_ref[...]
    m = jnp.max(logits, axis=1, keepdims=True)
    ie = lax.broadcasted_iota(jnp.int32, (GATE_BG, E), 1)
    top1 = jnp.min(jnp.where(logits == m, ie, E), axis=1)
    ie16 = lax.broadcasted_iota(jnp.int32, (GATE_BG, 16), 1)
    onehot = (ie16 == top1[:, None]).astype(jnp.float32)
    tril = (lax.broadcasted_iota(jnp.int32, (GATE_BG, GATE_BG), 0)
            >= lax.broadcasted_iota(jnp.int32, (GATE_BG, GATE_BG), 1)
            ).astype(jnp.float32)
    incl = lax.dot(tril, onehot, preferred_element_type=jnp.float32)
    base = base_ref[...]
    rank = jnp.sum(onehot * (incl + base), axis=1) - 1.0
    top1_ref[...] = top1
    rank_ref[...] = rank.astype(jnp.int32)
    newbase = base + jnp.sum(onehot, axis=0, keepdims=True)
    base_ref[...] = newbase

    @pl.when(b == GATE_NBLK - 1)
    def _():
        counts = newbase
        bpe = jnp.floor((counts + (BT - 1)) * (1.0 / BT))
        t16 = (lax.broadcasted_iota(jnp.int32, (16, 16), 0)
               <= lax.broadcasted_iota(jnp.int32, (16, 16), 1)
               ).astype(jnp.float32)
        cumb = lax.dot(bpe, t16, preferred_element_type=jnp.float32)
        starts_ref[...] = ((cumb - bpe) * BT).astype(jnp.int32)
        nonem = (bpe > 0.0).astype(jnp.float32)
        nonem_col = jnp.transpose(nonem)
        rank_col = jnp.transpose(
            lax.dot(nonem, t16, preferred_element_type=jnp.float32) - 1.0)
        startblk_col = jnp.transpose(cumb - bpe)
        e_col = lax.broadcasted_iota(jnp.int32, (16, NB_PAD), 0
                                     ).astype(jnp.float32)
        iota_b = lax.broadcasted_iota(jnp.int32, (16, NB_PAD), 1
                                      ).astype(jnp.float32)
        q = (startblk_col <= iota_b).astype(jnp.float32) * nonem_col
        bemap_ref[...] = jnp.max(e_col * q, axis=0,
                                 keepdims=True).astype(jnp.int32)
        ordmap_ref[...] = (jnp.sum(q, axis=0, keepdims=True)
                           - 1.0).astype(jnp.int32)
        e_col16 = lax.broadcasted_iota(jnp.int32, (16, 16), 0
                                       ).astype(jnp.float32)
        iota_k = lax.broadcasted_iota(jnp.int32, (16, 16), 1
                                      ).astype(jnp.float32)
        q2 = (rank_col <= iota_k).astype(jnp.float32) * nonem_col
        seg_ref[...] = jnp.max(e_col16 * q2, axis=0,
                               keepdims=True).astype(jnp.int32)


def _gate(x, Wg, bg):
    return pl.pallas_call(
        _gate_body,
        grid=(GATE_NBLK,),
        in_specs=[
            pl.BlockSpec((GATE_BG, DIM), lambda b: (b, 0)),
            pl.BlockSpec((DIM, E), lambda b: (0, 0)),
            pl.BlockSpec((1, E), lambda b: (0, 0)),
        ],
        out_specs=[
            pl.BlockSpec((GATE_BG,), lambda b: (b,)),
            pl.BlockSpec((GATE_BG,), lambda b: (b,)),
            pl.BlockSpec((1, 16), lambda b: (0, 0)),
            pl.BlockSpec((1, NB_PAD), lambda b: (0, 0)),
            pl.BlockSpec((1, NB_PAD), lambda b: (0, 0)),
            pl.BlockSpec((1, 16), lambda b: (0, 0)),
        ],
        out_shape=[
            jax.ShapeDtypeStruct((N,), jnp.int32),
            jax.ShapeDtypeStruct((N,), jnp.int32),
            jax.ShapeDtypeStruct((1, 16), jnp.int32),
            jax.ShapeDtypeStruct((1, NB_PAD), jnp.int32),
            jax.ShapeDtypeStruct((1, NB_PAD), jnp.int32),
            jax.ShapeDtypeStruct((1, 16), jnp.int32),
        ],
        scratch_shapes=[pltpu.VMEM((1, 16), jnp.float32)],
        compiler_params=pltpu.CompilerParams(
            dimension_semantics=("arbitrary",)),
    )(x, Wg, bg.reshape(1, E))


def _gmm_body(bm_ref, om_ref, seg_ref, xs_ref, b1_ref, b2_ref,
              w1_hbm, w2_hbm, out_ref, w1r, w2r, sem0, sem1):
    b = pl.program_id(0)
    k = om_ref[b]
    slot0 = lax.rem(k, 2) == 0
    sems = (sem0, sem1)

    def issue(e, s):
        pltpu.async_copy(w1_hbm.at[pl.ds(e, 1)], w1r.at[pl.ds(s, 1)],
                         sems[s])
        pltpu.async_copy(w2_hbm.at[pl.ds(e, 1)], w2r.at[pl.ds(s, 1)],
                         sems[s])

    def drain(s):
        pltpu.make_async_copy(w1_hbm.at[pl.ds(0, 1)],
                              w1r.at[pl.ds(s, 1)], sems[s]).wait()
        pltpu.make_async_copy(w2_hbm.at[pl.ds(0, 1)],
                              w2r.at[pl.ds(s, 1)], sems[s]).wait()

    nxt = seg_ref[jnp.minimum(k + 1, 15)]
    cur = seg_ref[k]

    @pl.when(b == 0)
    def _():
        issue(cur, 0)

        @pl.when(nxt != cur)
        def _():
            issue(nxt, 1)
        drain(0)

    @pl.when(jnp.logical_and(b > 0, k != om_ref[jnp.maximum(b - 1, 0)]))
    def _():
        @pl.when(slot0)
        def _():
            drain(0)

            @pl.when(nxt != cur)
            def _():
                issue(nxt, 1)

        @pl.when(jnp.logical_not(slot0))
        def _():
            drain(1)

            @pl.when(nxt != cur)
            def _():
                issue(nxt, 0)

    def compute(s):
        xb = xs_ref[...]
        h = jnp.maximum(
            lax.dot(xb, w1r[s], preferred_element_type=jnp.float32)
            + b1_ref[0], 0.0)
        out_ref[...] = (
            lax.dot(h, w2r[s], preferred_element_type=jnp.float32)
            + b2_ref[0] + xb)

    @pl.when(slot0)
    def _():
        compute(0)

    @pl.when(jnp.logical_not(slot0))
    def _():
        compute(1)


def _gmm(bemap, ordmap, seg, xs, W1, b1, W2, b2):
    grid_spec = pltpu.PrefetchScalarGridSpec(
        num_scalar_prefetch=3,
        grid=(NB,),
        in_specs=[
            pl.BlockSpec((BT, DIM), lambda b, bm, om, sg: (b, 0)),
            pl.BlockSpec((1, 1, HID), lambda b, bm, om, sg: (bm[b], 0, 0)),
            pl.BlockSpec((1, 1, DIM), lambda b, bm, om, sg: (bm[b], 0, 0)),
            pl.BlockSpec(memory_space=pl.ANY),
            pl.BlockSpec(memory_space=pl.ANY),
        ],
        out_specs=pl.BlockSpec((BT, DIM), lambda b, bm, om, sg: (b, 0)),
        scratch_shapes=[
            pltpu.VMEM((2, DIM, HID), jnp.float32),
            pltpu.VMEM((2, HID, DIM), jnp.float32),
            pltpu.SemaphoreType.DMA,
            pltpu.SemaphoreType.DMA,
        ],
    )
    return pl.pallas_call(
        _gmm_body,
        grid_spec=grid_spec,
        out_shape=jax.ShapeDtypeStruct((NPAD, DIM), jnp.float32),
        compiler_params=pltpu.CompilerParams(
            dimension_semantics=("arbitrary",)),
    )(bemap, ordmap, seg, xs, b1.reshape(E, 1, HID), b2.reshape(E, 1, DIM),
      W1, W2)


@functools.cache
def _sc_mesh():
    return plsc.VectorSubcoreMesh(core_axis_name="c", subcore_axis_name="s",
                                  num_cores=NC)


NCHUNK = (N // NW) // ROWS_PER_CHUNK


NRING = 3

_SC_SCRATCH = [
    [pltpu.VMEM((ROWS_PER_CHUNK,), jnp.int32)] * NCHUNK,
    pltpu.VMEM((N // NW,), jnp.int32),
    pltpu.VMEM((N // NW,), jnp.int32),
    pltpu.VMEM((16,), jnp.int32),
    pltpu.VMEM((NRING, ROWS_PER_CHUNK, DIM), jnp.float32),
    [pltpu.SemaphoreType.DMA] * NRING,
    [pltpu.SemaphoreType.DMA] * NRING,
    pltpu.SemaphoreType.DMA,
]


def _compute_pos(base, top1_hbm, rank_hbm, starts_hbm, idx_v, t_v, r_v, s_v,
                 psem):
    copies = [
        pltpu.async_copy(starts_hbm, s_v, psem),
        pltpu.async_copy(top1_hbm.at[pl.ds(base, N // NW)], t_v, psem),
        pltpu.async_copy(rank_hbm.at[pl.ds(base, N // NW)], r_v, psem),
    ]
    for cp in copies:
        cp.wait()
    s_vec = s_v[...]
    for c in range(NCHUNK):
        for g in range(ROWS_PER_CHUNK // 16):
            o = c * ROWS_PER_CHUNK + g * 16
            sv = s_vec.at[t_v[pl.ds(o, 16)]].get(mode="promise_in_bounds")
            idx_v[c][pl.ds(g * 16, 16)] = sv + r_v[pl.ds(o, 16)]


def _sc_scatter_rows(x, top1, rank, starts):
    @functools.partial(
        pl.kernel,
        mesh=_sc_mesh(),
        out_type=jax.ShapeDtypeStruct((NPAD, DIM), jnp.float32),
        scratch_types=_SC_SCRATCH,
    )
    def k(x_hbm, top1_hbm, rank_hbm, starts_hbm, xs_hbm,
          idx_v, t_v, r_v, s_v, rows_v, rsem, ssem, psem):
        wid = lax.axis_index("s") * NC + lax.axis_index("c")
        base = wid * (N // NW)
        reads, writes = [None] * NCHUNK, [None] * NCHUNK

        def start_read(c):
            p = c % NRING
            reads[c] = pltpu.async_copy(
                x_hbm.at[pl.ds(base + c * ROWS_PER_CHUNK, ROWS_PER_CHUNK)],
                rows_v.at[p], rsem[p])

        for c in range(NRING):
            start_read(c)
        _compute_pos(base, top1_hbm, rank_hbm, starts_hbm, idx_v,
                     t_v, r_v, s_v, psem)
        for c in range(NCHUNK):
            p = c % NRING
            reads[c].wait()
            writes[c] = pltpu.async_copy(
                rows_v.at[p], xs_hbm.at[idx_v[c]], ssem[p])
            if c + NRING < NCHUNK:
                writes[c].wait()
                start_read(c + NRING)
        for c in range(max(0, NCHUNK - NRING), NCHUNK):
            writes[c].wait()

    return k(x, top1, rank, starts)


def _sc_gather_rows(ys, top1, rank, starts):
    @functools.partial(
        pl.kernel,
        mesh=_sc_mesh(),
        out_type=jax.ShapeDtypeStruct((N, DIM), jnp.float32),
        scratch_types=_SC_SCRATCH,
    )
    def k(ys_hbm, top1_hbm, rank_hbm, starts_hbm, out_hbm,
          idx_v, t_v, r_v, s_v, rows_v, rsem, ssem, psem):
        wid = lax.axis_index("s") * NC + lax.axis_index("c")
        base = wid * (N // NW)
        reads, writes = [None] * NCHUNK, [None] * NCHUNK

        def start_read(c):
            p = c % NRING
            reads[c] = pltpu.async_copy(
                ys_hbm.at[idx_v[c]], rows_v.at[p], rsem[p])

        _compute_pos(base, top1_hbm, rank_hbm, starts_hbm, idx_v,
                     t_v, r_v, s_v, psem)
        for c in range(NRING):
            start_read(c)
        for c in range(NCHUNK):
            p = c % NRING
            reads[c].wait()
            writes[c] = pltpu.async_copy(
                rows_v.at[p],
                out_hbm.at[pl.ds(base + c * ROWS_PER_CHUNK, ROWS_PER_CHUNK)],
                ssem[p])
            if c + NRING < NCHUNK:
                writes[c].wait()
                start_read(c + NRING)
        for c in range(max(0, NCHUNK - NRING), NCHUNK):
            writes[c].wait()

    return k(ys, top1, rank, starts)


def kernel(x, Wg, bg, W1, b1, W2, b2):
    top1, rank, starts16, bemap, ordmap, seg = _gate(x, Wg, bg)
    starts = starts16.reshape(16)

    xs = _sc_scatter_rows(x, top1, rank, starts)
    ys = _gmm(bemap.reshape(NB_PAD), ordmap.reshape(NB_PAD),
              seg.reshape(16), xs, W1, b1, W2, b2)
    return _sc_gather_rows(ys, top1, rank, starts)


# speedup vs baseline: 1.1081x; 1.1081x over previous
import functools

import jax
import jax.numpy as jnp
from jax import lax
from jax.experimental import pallas as pl
from jax.experimental.pallas import tpu as pltpu
from jax.experimental.pallas import tpu_sc as plsc

N = 4096
DIM = 1024
E = 8
HID = 1536

BT = 384
NB = N // BT + E
NB_PAD = 128
NPAD = NB * BT

GATE_BG = 512
GATE_NBLK = N // GATE_BG

NC = 2
NS = 16
NW = NC * NS
ROWS_PER_CHUNK = 32


def _gate_body(x_ref, wg_ref, bg_ref, top1_ref, rank_ref, starts_ref,
               bemap_ref, ordmap_ref, seg_ref, base_ref):
    b = pl.program_id(0)

    @pl.when(b == 0)
    def _():
        base_ref[...] = jnp.zeros((1, 16), jnp.float32)

    logits = lax.dot(x_ref[...], wg_ref[...],
                     preferred_element_type=jnp.float32) + bg_ref[...]
    m = jnp.max(logits, axis=1, keepdims=True)
    ie = lax.broadcasted_iota(jnp.int32, (GATE_BG, E), 1)
    top1 = jnp.min(jnp.where(logits == m, ie, E), axis=1)
    ie16 = lax.broadcasted_iota(jnp.int32, (GATE_BG, 16), 1)
    onehot = (ie16 == top1[:, None]).astype(jnp.float32)
    tril = (lax.broadcasted_iota(jnp.int32, (GATE_BG, GATE_BG), 0)
            >= lax.broadcasted_iota(jnp.int32, (GATE_BG, GATE_BG), 1)
            ).astype(jnp.float32)
    incl = lax.dot(tril, onehot, preferred_element_type=jnp.float32)
    base = base_ref[...]
    rank = jnp.sum(onehot * (incl + base), axis=1) - 1.0
    top1_ref[...] = top1
    rank_ref[...] = rank.astype(jnp.int32)
    newbase = base + jnp.sum(onehot, axis=0, keepdims=True)
    base_ref[...] = newbase

    @pl.when(b == GATE_NBLK - 1)
    def _():
        counts = newbase
        bpe = jnp.floor((counts + (BT - 1)) * (1.0 / BT))
        t16 = (lax.broadcasted_iota(jnp.int32, (16, 16), 0)
               <= lax.broadcasted_iota(jnp.int32, (16, 16), 1)
               ).astype(jnp.float32)
        cumb = lax.dot(bpe, t16, preferred_element_type=jnp.float32)
        starts_ref[...] = ((cumb - bpe) * BT).astype(jnp.int32)
        nonem = (bpe > 0.0).astype(jnp.float32)
        nonem_col = jnp.transpose(nonem)
        rank_col = jnp.transpose(
            lax.dot(nonem, t16, preferred_element_type=jnp.float32) - 1.0)
        startblk_col = jnp.transpose(cumb - bpe)
        e_col = lax.broadcasted_iota(jnp.int32, (16, NB_PAD), 0
                                     ).astype(jnp.float32)
        iota_b = lax.broadcasted_iota(jnp.int32, (16, NB_PAD), 1
                                      ).astype(jnp.float32)
        q = (startblk_col <= iota_b).astype(jnp.float32) * nonem_col
        bemap_ref[...] = jnp.max(e_col * q, axis=0,
                                 keepdims=True).astype(jnp.int32)
        ordmap_ref[...] = (jnp.sum(q, axis=0, keepdims=True)
                           - 1.0).astype(jnp.int32)
        e_col16 = lax.broadcasted_iota(jnp.int32, (16, 16), 0
                                       ).astype(jnp.float32)
        iota_k = lax.broadcasted_iota(jnp.int32, (16, 16), 1
                                      ).astype(jnp.float32)
        q2 = (rank_col <= iota_k).astype(jnp.float32) * nonem_col
        seg_ref[...] = jnp.max(e_col16 * q2, axis=0,
                               keepdims=True).astype(jnp.int32)


def _gate(x, Wg, bg):
    return pl.pallas_call(
        _gate_body,
        grid=(GATE_NBLK,),
        in_specs=[
            pl.BlockSpec((GATE_BG, DIM), lambda b: (b, 0)),
            pl.BlockSpec((DIM, E), lambda b: (0, 0)),
            pl.BlockSpec((1, E), lambda b: (0, 0)),
        ],
        out_specs=[
            pl.BlockSpec((GATE_BG,), lambda b: (b,)),
            pl.BlockSpec((GATE_BG,), lambda b: (b,)),
            pl.BlockSpec((1, 16), lambda b: (0, 0)),
            pl.BlockSpec((1, NB_PAD), lambda b: (0, 0)),
            pl.BlockSpec((1, NB_PAD), lambda b: (0, 0)),
            pl.BlockSpec((1, 16), lambda b: (0, 0)),
        ],
        out_shape=[
            jax.ShapeDtypeStruct((N,), jnp.int32),
            jax.ShapeDtypeStruct((N,), jnp.int32),
            jax.ShapeDtypeStruct((1, 16), jnp.int32),
            jax.ShapeDtypeStruct((1, NB_PAD), jnp.int32),
            jax.ShapeDtypeStruct((1, NB_PAD), jnp.int32),
            jax.ShapeDtypeStruct((1, 16), jnp.int32),
        ],
        scratch_shapes=[pltpu.VMEM((1, 16), jnp.float32)],
        compiler_params=pltpu.CompilerParams(
            dimension_semantics=("arbitrary",)),
    )(x, Wg, bg.reshape(1, E))


def _gmm_body(bm_ref, om_ref, seg_ref, xs_ref, b1_ref, b2_ref,
              w1_hbm, w2_hbm, out_ref, w1r, w2r, sem0, sem1):
    b = pl.program_id(0)
    k = om_ref[b]
    slot0 = lax.rem(k, 2) == 0
    sems = (sem0, sem1)

    def issue(e, s):
        pltpu.async_copy(w1_hbm.at[pl.ds(e, 1)], w1r.at[pl.ds(s, 1)],
                         sems[s])
        pltpu.async_copy(w2_hbm.at[pl.ds(e, 1)], w2r.at[pl.ds(s, 1)],
                         sems[s])

    def drain(s):
        pltpu.make_async_copy(w1_hbm.at[pl.ds(0, 1)],
                              w1r.at[pl.ds(s, 1)], sems[s]).wait()
        pltpu.make_async_copy(w2_hbm.at[pl.ds(0, 1)],
                              w2r.at[pl.ds(s, 1)], sems[s]).wait()

    nxt = seg_ref[jnp.minimum(k + 1, 15)]
    cur = seg_ref[k]

    @pl.when(b == 0)
    def _():
        issue(cur, 0)

        @pl.when(nxt != cur)
        def _():
            issue(nxt, 1)
        drain(0)

    @pl.when(jnp.logical_and(b > 0, k != om_ref[jnp.maximum(b - 1, 0)]))
    def _():
        @pl.when(slot0)
        def _():
            drain(0)

            @pl.when(nxt != cur)
            def _():
                issue(nxt, 1)

        @pl.when(jnp.logical_not(slot0))
        def _():
            drain(1)

            @pl.when(nxt != cur)
            def _():
                issue(nxt, 0)

    def compute(s):
        xb = xs_ref[...]
        h = jnp.maximum(
            lax.dot(xb, w1r[s], preferred_element_type=jnp.float32)
            + b1_ref[0], 0.0)
        out_ref[...] = (
            lax.dot(h, w2r[s], preferred_element_type=jnp.float32)
            + b2_ref[0] + xb)

    @pl.when(slot0)
    def _():
        compute(0)

    @pl.when(jnp.logical_not(slot0))
    def _():
        compute(1)


def _gmm(bemap, ordmap, seg, xs, W1, b1, W2, b2):
    grid_spec = pltpu.PrefetchScalarGridSpec(
        num_scalar_prefetch=3,
        grid=(NB,),
        in_specs=[
            pl.BlockSpec((BT, DIM), lambda b, bm, om, sg: (b, 0)),
            pl.BlockSpec((1, 1, HID), lambda b, bm, om, sg: (bm[b], 0, 0)),
            pl.BlockSpec((1, 1, DIM), lambda b, bm, om, sg: (bm[b], 0, 0)),
            pl.BlockSpec(memory_space=pl.ANY),
            pl.BlockSpec(memory_space=pl.ANY),
        ],
        out_specs=pl.BlockSpec((BT, DIM), lambda b, bm, om, sg: (b, 0)),
        scratch_shapes=[
            pltpu.VMEM((2, DIM, HID), jnp.float32),
            pltpu.VMEM((2, HID, DIM), jnp.float32),
            pltpu.SemaphoreType.DMA,
            pltpu.SemaphoreType.DMA,
        ],
    )
    return pl.pallas_call(
        _gmm_body,
        grid_spec=grid_spec,
        out_shape=jax.ShapeDtypeStruct((NPAD, DIM), jnp.float32),
        compiler_params=pltpu.CompilerParams(
            dimension_semantics=("arbitrary",)),
    )(bemap, ordmap, seg, xs, b1.reshape(E, 1, HID), b2.reshape(E, 1, DIM),
      W1, W2)


@functools.cache
def _sc_mesh():
    return plsc.VectorSubcoreMesh(core_axis_name="c", subcore_axis_name="s",
                                  num_cores=NC)


NCHUNK = (N // NW) // ROWS_PER_CHUNK


NRING = 3

_SC_SCRATCH = [
    [pltpu.VMEM((ROWS_PER_CHUNK,), jnp.int32)] * NCHUNK,
    pltpu.VMEM((N // NW,), jnp.int32),
    pltpu.VMEM((N // NW,), jnp.int32),
    pltpu.VMEM((16,), jnp.int32),
    pltpu.VMEM((NRING, ROWS_PER_CHUNK, DIM), jnp.float32),
    [pltpu.SemaphoreType.DMA] * NRING,
    [pltpu.SemaphoreType.DMA] * NRING,
    pltpu.SemaphoreType.DMA,
]


def _compute_pos(base, top1_hbm, rank_hbm, starts_hbm, idx_v, t_v, r_v, s_v,
                 psem):
    copies = [
        pltpu.async_copy(starts_hbm, s_v, psem),
        pltpu.async_copy(top1_hbm.at[pl.ds(base, N // NW)], t_v, psem),
        pltpu.async_copy(rank_hbm.at[pl.ds(base, N // NW)], r_v, psem),
    ]
    for cp in copies:
        cp.wait()
    s_vec = s_v[...]
    for c in range(NCHUNK):
        for g in range(ROWS_PER_CHUNK // 16):
            o = c * ROWS_PER_CHUNK + g * 16
            sv = s_vec.at[t_v[pl.ds(o, 16)]].get(mode="promise_in_bounds")
            idx_v[c][pl.ds(g * 16, 16)] = sv + r_v[pl.ds(o, 16)]


def _sc_scatter_rows(x, top1, rank, starts):
    @functools.partial(
        pl.kernel,
        mesh=_sc_mesh(),
        out_type=jax.ShapeDtypeStruct((NPAD, DIM), jnp.float32),
        scratch_types=_SC_SCRATCH,
    )
    def k(x_hbm, top1_hbm, rank_hbm, starts_hbm, xs_hbm,
          idx_v, t_v, r_v, s_v, rows_v, rsem, ssem, psem):
        wid = lax.axis_index("s") * NC + lax.axis_index("c")
        base = wid * (N // NW)
        reads, writes = [None] * NCHUNK, [None] * NCHUNK

        def start_read(c):
            p = c % NRING
            reads[c] = pltpu.async_copy(
                x_hbm.at[pl.ds(base + c * ROWS_PER_CHUNK, ROWS_PER_CHUNK)],
                rows_v.at[p], rsem[p])

        for c in range(NRING):
            start_read(c)
        _compute_pos(base, top1_hbm, rank_hbm, starts_hbm, idx_v,
                     t_v, r_v, s_v, psem)
        for c in range(NCHUNK):
            p = c % NRING
            reads[c].wait()
            writes[c] = pltpu.async_copy(
                rows_v.at[p], xs_hbm.at[idx_v[c]], ssem[p])
            if c + NRING < NCHUNK:
                writes[c].wait()
                start_read(c + NRING)
        for c in range(max(0, NCHUNK - NRING), NCHUNK):
            writes[c].wait()

    return k(x, top1, rank, starts)


def _sc_gather_rows(ys, top1, rank, starts):
    @functools.partial(
        pl.kernel,
        mesh=_sc_mesh(),
        out_type=jax.ShapeDtypeStruct((N, DIM), jnp.float32),
        scratch_types=_SC_SCRATCH,
    )
    def k(ys_hbm, top1_hbm, rank_hbm, starts_hbm, out_hbm,
          idx_v, t_v, r_v, s_v, rows_v, rsem, ssem, psem):
        wid = lax.axis_index("s") * NC + lax.axis_index("c")
        base = wid * (N // NW)
        reads, writes = [None] * NCHUNK, [None] * NCHUNK

        def start_read(c):
            p = c % NRING
            reads[c] = pltpu.async_copy(
                ys_hbm.at[idx_v[c]], rows_v.at[p], rsem[p])

        _compute_pos(base, top1_hbm, rank_hbm, starts_hbm, idx_v,
                     t_v, r_v, s_v, psem)
        for c in range(NRING):
            start_read(c)
        for c in range(NCHUNK):
            p = c % NRING
            reads[c].wait()
            writes[c] = pltpu.async_copy(
                rows_v.at[p],
                out_hbm.at[pl.ds(base + c * ROWS_PER_CHUNK, ROWS_PER_CHUNK)],
                ssem[p])
            if c + NRING < NCHUNK:
                writes[c].wait()
                start_read(c + NRING)
        for c in range(max(0, NCHUNK - NRING), NCHUNK):
            writes[c].wait()

    return k(ys, top1, rank, starts)


def kernel(x, Wg, bg, W1, b1, W2, b2):
    top1, rank, starts16, bemap, ordmap, seg = _gate(x, Wg, bg)
    starts = starts16.reshape(16)

    xs = _sc_scatter_rows(x, top1, rank, starts)
    ys = _gmm(bemap.reshape(NB_PAD), ordmap.reshape(NB_PAD),
              seg.reshape(16), xs, W1, b1, W2, b2)
    return _sc_gather_rows(ys, top1, rank, starts)
